# baseline (speedup 1.0000x reference)
import functools

import jax
import jax.numpy as jnp
from jax import lax
from jax.experimental import pallas as pl
from jax.experimental.pallas import tpu as pltpu
from jax.experimental.pallas import tpu_sc as plsc

N_NODES = 50000
D = 128
N_EDGES = 400000

NUM_TILES = 16
CHUNK = 4096
N_CHUNKS = 16
N_PAD = N_CHUNKS * CHUNK
STRIPE = CHUNK // NUM_TILES
TRASH = CHUNK
G = 64
EP = 25088
E_PAD = NUM_TILES * EP


def _sc_segment_sums(x_paper, x_author, src_c, dst_c, src_w, dst_w, z_rows,
                     ones_rows):
  mesh = plsc.VectorSubcoreMesh(core_axis_name="c", subcore_axis_name="s")
  fdt = jnp.float32

  @functools.partial(
      pl.kernel,
      out_type=[
          jax.ShapeDtypeStruct((N_PAD, D), fdt),
          jax.ShapeDtypeStruct((N_PAD, D), fdt),
          jax.ShapeDtypeStruct((N_PAD, D), fdt),
          jax.ShapeDtypeStruct((N_PAD, D), fdt),
      ],
      mesh=mesh,
      scratch_types=[
          pltpu.VMEM_SHARED((CHUNK + 1, D), fdt),
          pltpu.VMEM_SHARED((CHUNK + 1, D), fdt),
          pltpu.VMEM((G,), jnp.int32),
          pltpu.VMEM((G,), jnp.int32),
          pltpu.VMEM((G,), jnp.int32),
          pltpu.VMEM((G, D), fdt),
          pltpu.VMEM((G, D), fdt),
          pltpu.VMEM((G,), jnp.int32),
          pltpu.VMEM((G, D), fdt),
          pltpu.SemaphoreType.DMA,
      ],
  )
  def k(xp_hbm, xa_hbm, srcc_hbm, dstc_hbm, srcw_hbm, dstw_hbm, zr_hbm,
        ones_hbm,
        sums_c_hbm, cnt_c_hbm, sums_w_hbm, cnt_w_hbm,
        spm_sums, spm_cnt, src_v, dst_v, ldst_v, rows_v, ones_vm,
        pidx_v, stg_v, sem):
    core = lax.axis_index("c")
    wid = lax.axis_index("s")
    i16 = lax.iota(jnp.int32, 16)
    pltpu.sync_copy(ones_hbm, ones_vm)

    def set_piece_indices(base):
      for j in range(G // 16):
        pidx_v[pl.ds(j * 16, 16)] = base + j * 16 + i16

    for table, src_hbm, dst_hbm, sums_out, cnt_out in (
        (xp_hbm, srcc_hbm, dstc_hbm, sums_c_hbm, cnt_c_hbm),
        (xa_hbm, srcw_hbm, dstw_hbm, sums_w_hbm, cnt_w_hbm),
    ):
      for lc in range(N_CHUNKS // 2):
        chunk = (N_CHUNKS // 2) * core + lc
        lo = chunk * CHUNK
        hi = lo + CHUNK

        pltpu.sync_copy(zr_hbm, stg_v)
        for off in range(0, STRIPE, G):
          set_piece_indices(wid * STRIPE + off)
          pltpu.sync_copy(stg_v, spm_sums.at[pidx_v])
          pltpu.sync_copy(stg_v, spm_cnt.at[pidx_v])
        plsc.subcore_barrier()

        @pl.when(lo < N_NODES)
        def _():
          def step(g, _):
            base = wid * EP + g * G
            pltpu.sync_copy(src_hbm.at[pl.ds(base, G)], src_v)
            pltpu.sync_copy(dst_hbm.at[pl.ds(base, G)], dst_v)
            for j in range(G // 16):
              d = dst_v[pl.ds(j * 16, 16)]
              m = (d >= lo) & (d < hi)
              ldst_v[pl.ds(j * 16, 16)] = jnp.where(m, d - lo, TRASH)
            pltpu.async_copy(table.at[src_v], rows_v, sem).wait()
            pltpu.sync_copy(rows_v, spm_sums.at[ldst_v], add=True)
            pltpu.sync_copy(ones_vm, spm_cnt.at[ldst_v], add=True)
            return 0

          lax.fori_loop(0, EP // G, step, 0)

        plsc.subcore_barrier()

        row0 = lo + wid * STRIPE
        for off in range(0, STRIPE, G):
          set_piece_indices(wid * STRIPE + off)
          pltpu.sync_copy(spm_sums.at[pidx_v], stg_v)
          pltpu.sync_copy(stg_v, sums_out.at[pl.ds(row0 + off, G)])
          pltpu.sync_copy(spm_cnt.at[pidx_v], stg_v)
          pltpu.sync_copy(stg_v, cnt_out.at[pl.ds(row0 + off, G)])
        plsc.subcore_barrier()

  return k(x_paper, x_author, src_c, dst_c, src_w, dst_w, z_rows, ones_rows)


def _tc_combine_body(xp_ref, xa_ref, sc_ref, cc_ref, sw_ref, cw_ref,
                     wrp_ref, brp_ref, wc_ref, ww_ref, wra_ref, bra_ref,
                     op_ref, oa_ref):
  dot = functools.partial(jnp.dot, preferred_element_type=jnp.float32,
                          precision=lax.Precision.HIGHEST)
  inv_c = 1.0 / jnp.maximum(cc_ref[...], 1.0)
  inv_w = 1.0 / jnp.maximum(cw_ref[...], 1.0)
  agg_c = sc_ref[...] * inv_c
  agg_w = sw_ref[...] * inv_w
  op_ref[...] = (dot(xp_ref[...], wrp_ref[...]) + dot(agg_c, wc_ref[...])
                 + dot(agg_w, ww_ref[...]) + brp_ref[...])
  oa_ref[...] = dot(xa_ref[...], wra_ref[...]) + bra_ref[...]


def _tc_combine(xp, xa, sums_c, cnt_c, sums_w, cnt_w,
                W_root_p, b_root_p, W_c, W_w, W_root_a, b_root_a):
  blk = 512
  grid = (pl.cdiv(N_NODES, blk),)
  row_spec = lambda w: pl.BlockSpec((blk, w), lambda i: (i, 0))
  full_spec = pl.BlockSpec((D, D), lambda i: (0, 0))
  bias_spec = pl.BlockSpec((1, D), lambda i: (0, 0))
  return pl.pallas_call(
      _tc_combine_body,
      grid=grid,
      in_specs=[row_spec(D), row_spec(D), row_spec(D), row_spec(D),
                row_spec(D), row_spec(D),
                full_spec, bias_spec, full_spec, full_spec,
                full_spec, bias_spec],
      out_specs=[row_spec(D), row_spec(D)],
      out_shape=[jax.ShapeDtypeStruct((N_NODES, D), jnp.float32),
                 jax.ShapeDtypeStruct((N_NODES, D), jnp.float32)],
  )(xp, xa, sums_c, cnt_c, sums_w, cnt_w,
    W_root_p, b_root_p.reshape(1, D), W_c, W_w,
    W_root_a, b_root_a.reshape(1, D))


@jax.jit
def kernel(x_paper, x_author, edge_index_cites, edge_index_writes,
           W_cites, W_writes, W_root_paper, b_root_paper,
           W_root_author, b_root_author):
  pad = E_PAD - N_EDGES
  src_c = jnp.pad(edge_index_cites[0], (0, pad))
  dst_c = jnp.pad(edge_index_cites[1], (0, pad), constant_values=-1)
  src_w = jnp.pad(edge_index_writes[0], (0, pad))
  dst_w = jnp.pad(edge_index_writes[1], (0, pad), constant_values=-1)

  z_rows = jnp.zeros((G, D), jnp.float32)
  ones_rows = jnp.ones((G, D), jnp.float32)

  sums_c, cnt_c, sums_w, cnt_w = _sc_segment_sums(
      x_paper, x_author, src_c, dst_c, src_w, dst_w, z_rows, ones_rows)

  out_paper, out_author = _tc_combine(
      x_paper, x_author,
      sums_c[:N_NODES], cnt_c[:N_NODES], sums_w[:N_NODES], cnt_w[:N_NODES],
      W_root_paper, b_root_paper, W_cites, W_writes,
      W_root_author, b_root_author)
  return (out_paper, out_author)


# speedup vs baseline: 1.3084x; 1.3084x over previous
import functools

import jax
import jax.numpy as jnp
from jax import lax
from jax.experimental import pallas as pl
from jax.experimental.pallas import tpu as pltpu
from jax.experimental.pallas import tpu_sc as plsc

N_NODES = 50000
D = 128
N_EDGES = 400000

NUM_TILES = 16
CHUNK = 4096
N_CHUNKS = 16
N_PAD = N_CHUNKS * CHUNK
STRIPE = CHUNK // NUM_TILES
TRASH = CHUNK
G = 128
EP = 25088
E_PAD = NUM_TILES * EP


def _sc_segment_sums(x_paper, x_author, src_c, dst_c, src_w, dst_w, z_rows,
                     ones_rows):
  mesh = plsc.VectorSubcoreMesh(core_axis_name="c", subcore_axis_name="s")
  fdt = jnp.float32

  @functools.partial(
      pl.kernel,
      out_type=[
          jax.ShapeDtypeStruct((N_PAD, D), fdt),
          jax.ShapeDtypeStruct((N_PAD, D), fdt),
          jax.ShapeDtypeStruct((N_PAD, D), fdt),
          jax.ShapeDtypeStruct((N_PAD, D), fdt),
      ],
      mesh=mesh,
      scratch_types=[
          pltpu.VMEM_SHARED((CHUNK + 1, D), fdt),
          pltpu.VMEM_SHARED((CHUNK + 1, D), fdt),
          pltpu.VMEM((G,), jnp.int32),
          pltpu.VMEM((G,), jnp.int32),
          pltpu.VMEM((G,), jnp.int32),
          pltpu.VMEM((G, D), fdt),
          pltpu.VMEM((G, D), fdt),
          pltpu.VMEM((G,), jnp.int32),
          pltpu.VMEM((G, D), fdt),
          pltpu.SemaphoreType.DMA,
      ],
  )
  def k(xp_hbm, xa_hbm, srcc_hbm, dstc_hbm, srcw_hbm, dstw_hbm, zr_hbm,
        ones_hbm,
        sums_c_hbm, cnt_c_hbm, sums_w_hbm, cnt_w_hbm,
        spm_sums, spm_cnt, src_v, dst_v, ldst_v, rows_v, ones_vm,
        pidx_v, stg_v, sem):
    core = lax.axis_index("c")
    wid = lax.axis_index("s")
    i16 = lax.iota(jnp.int32, 16)
    pltpu.sync_copy(ones_hbm, ones_vm)

    def set_piece_indices(base):
      for j in range(G // 16):
        pidx_v[pl.ds(j * 16, 16)] = base + j * 16 + i16

    for table, src_hbm, dst_hbm, sums_out, cnt_out in (
        (xp_hbm, srcc_hbm, dstc_hbm, sums_c_hbm, cnt_c_hbm),
        (xa_hbm, srcw_hbm, dstw_hbm, sums_w_hbm, cnt_w_hbm),
    ):
      for lc in range(N_CHUNKS // 2):
        chunk = (N_CHUNKS // 2) * core + lc
        lo = chunk * CHUNK
        hi = lo + CHUNK

        pltpu.sync_copy(zr_hbm, stg_v)
        for off in range(0, STRIPE, G):
          set_piece_indices(wid * STRIPE + off)
          pltpu.sync_copy(stg_v, spm_sums.at[pidx_v])
          pltpu.sync_copy(stg_v, spm_cnt.at[pidx_v])
        plsc.subcore_barrier()

        @pl.when(lo < N_NODES)
        def _():
          def step(g, _):
            base = wid * EP + g * G
            pltpu.sync_copy(src_hbm.at[pl.ds(base, G)], src_v)
            pltpu.sync_copy(dst_hbm.at[pl.ds(base, G)], dst_v)
            for j in range(G // 16):
              d = dst_v[pl.ds(j * 16, 16)]
              m = (d >= lo) & (d < hi)
              ldst_v[pl.ds(j * 16, 16)] = jnp.where(m, d - lo, TRASH)
            pltpu.async_copy(table.at[src_v], rows_v, sem).wait()
            pltpu.sync_copy(rows_v, spm_sums.at[ldst_v], add=True)
            pltpu.sync_copy(ones_vm, spm_cnt.at[ldst_v], add=True)
            return 0

          lax.fori_loop(0, EP // G, step, 0)

        plsc.subcore_barrier()

        row0 = lo + wid * STRIPE
        for off in range(0, STRIPE, G):
          set_piece_indices(wid * STRIPE + off)
          pltpu.sync_copy(spm_sums.at[pidx_v], stg_v)
          pltpu.sync_copy(stg_v, sums_out.at[pl.ds(row0 + off, G)])
          pltpu.sync_copy(spm_cnt.at[pidx_v], stg_v)
          pltpu.sync_copy(stg_v, cnt_out.at[pl.ds(row0 + off, G)])
        plsc.subcore_barrier()

  return k(x_paper, x_author, src_c, dst_c, src_w, dst_w, z_rows, ones_rows)


def _tc_combine_body(xp_ref, xa_ref, sc_ref, cc_ref, sw_ref, cw_ref,
                     wrp_ref, brp_ref, wc_ref, ww_ref, wra_ref, bra_ref,
                     op_ref, oa_ref):
  dot = functools.partial(jnp.dot, preferred_element_type=jnp.float32,
                          precision=lax.Precision.HIGHEST)
  inv_c = 1.0 / jnp.maximum(cc_ref[...], 1.0)
  inv_w = 1.0 / jnp.maximum(cw_ref[...], 1.0)
  agg_c = sc_ref[...] * inv_c
  agg_w = sw_ref[...] * inv_w
  op_ref[...] = (dot(xp_ref[...], wrp_ref[...]) + dot(agg_c, wc_ref[...])
                 + dot(agg_w, ww_ref[...]) + brp_ref[...])
  oa_ref[...] = dot(xa_ref[...], wra_ref[...]) + bra_ref[...]


def _tc_combine(xp, xa, sums_c, cnt_c, sums_w, cnt_w,
                W_root_p, b_root_p, W_c, W_w, W_root_a, b_root_a):
  blk = 512
  grid = (pl.cdiv(N_NODES, blk),)
  row_spec = lambda w: pl.BlockSpec((blk, w), lambda i: (i, 0))
  full_spec = pl.BlockSpec((D, D), lambda i: (0, 0))
  bias_spec = pl.BlockSpec((1, D), lambda i: (0, 0))
  return pl.pallas_call(
      _tc_combine_body,
      grid=grid,
      in_specs=[row_spec(D), row_spec(D), row_spec(D), row_spec(D),
                row_spec(D), row_spec(D),
                full_spec, bias_spec, full_spec, full_spec,
                full_spec, bias_spec],
      out_specs=[row_spec(D), row_spec(D)],
      out_shape=[jax.ShapeDtypeStruct((N_NODES, D), jnp.float32),
                 jax.ShapeDtypeStruct((N_NODES, D), jnp.float32)],
  )(xp, xa, sums_c, cnt_c, sums_w, cnt_w,
    W_root_p, b_root_p.reshape(1, D), W_c, W_w,
    W_root_a, b_root_a.reshape(1, D))


@jax.jit
def kernel(x_paper, x_author, edge_index_cites, edge_index_writes,
           W_cites, W_writes, W_root_paper, b_root_paper,
           W_root_author, b_root_author):
  pad = E_PAD - N_EDGES
  src_c = jnp.pad(edge_index_cites[0], (0, pad))
  dst_c = jnp.pad(edge_index_cites[1], (0, pad), constant_values=-1)
  src_w = jnp.pad(edge_index_writes[0], (0, pad))
  dst_w = jnp.pad(edge_index_writes[1], (0, pad), constant_values=-1)

  z_rows = jnp.zeros((G, D), jnp.float32)
  ones_rows = jnp.ones((G, D), jnp.float32)

  sums_c, cnt_c, sums_w, cnt_w = _sc_segment_sums(
      x_paper, x_author, src_c, dst_c, src_w, dst_w, z_rows, ones_rows)

  out_paper, out_author = _tc_combine(
      x_paper, x_author,
      sums_c[:N_NODES], cnt_c[:N_NODES], sums_w[:N_NODES], cnt_w[:N_NODES],
      W_root_paper, b_root_paper, W_cites, W_writes,
      W_root_author, b_root_author)
  return (out_paper, out_author)
